# baseline (speedup 1.0000x reference)
import functools

import jax
import jax.numpy as jnp
from jax import lax
from jax.experimental import pallas as pl
from jax.experimental.pallas import tpu as pltpu
from jax.experimental.pallas import tpu_sc as plsc

NC = 2
NS = 16
CHUNK = 128

_MESH = dict(core_axis_name="c", subcore_axis_name="s", num_cores=NC,
             num_subcores=NS)


def _make_layer_kernel(n, e, dh, edge_split, const_rows=False):
    nchunks = e // CHUNK
    rpt = n // NS
    nw = NC * NS if (edge_split or const_rows) else NS
    K = nchunks // nw
    assert K * nw == nchunks and K >= 4 and K % 2 == 0

    scratch = [
        pltpu.VMEM((CHUNK,), jnp.int32),
        pltpu.VMEM((CHUNK,), jnp.int32),
        pltpu.VMEM((CHUNK,), jnp.int32),
        pltpu.VMEM((CHUNK,), jnp.int32),
        pltpu.VMEM((CHUNK,), jnp.int32),
        pltpu.VMEM((CHUNK,), jnp.int32),
        pltpu.VMEM((CHUNK, dh), jnp.float32),
        pltpu.VMEM((CHUNK, dh), jnp.float32),
        pltpu.VMEM_SHARED((n, dh), jnp.float32),
        pltpu.SemaphoreType.DMA,
        pltpu.SemaphoreType.DMA,
        pltpu.SemaphoreType.DMA,
        pltpu.SemaphoreType.DMA,
    ]

    @functools.partial(
        pl.kernel,
        out_type=jax.ShapeDtypeStruct((NC, n, dh), jnp.float32),
        mesh=plsc.VectorSubcoreMesh(**_MESH),
        scratch_types=scratch,
    )
    def layer_kernel(g_hbm, src_hbm, dst_hbm, zeros_hbm, out_hbm,
                     srcb0, srcb1, srca0, srca1, dstb0, dstb1, rows0, rows1,
                     acc, isem0, isem1, gsem0, gsem1):
        c = lax.axis_index("c")
        s = lax.axis_index("s")
        if edge_split or const_rows:
            w = c * NS + s
        else:
            w = s
        base = jnp.int32(0) if (edge_split or const_rows) else c * n
        bufs = ((srcb0, srca0, dstb0, rows0, isem0, gsem0),
                (srcb1, srca1, dstb1, rows1, isem1, gsem1))

        if const_rows:
            pltpu.sync_copy(g_hbm, rows0)
            pltpu.sync_copy(zeros_hbm.at[pl.ds(s * rpt, rpt)],
                            acc.at[pl.ds(s * rpt, rpt)])
        elif edge_split:
            @pl.when(c == 0)
            def _():
                pltpu.sync_copy(g_hbm.at[pl.ds(s * rpt, rpt)],
                                acc.at[pl.ds(s * rpt, rpt)])

            @pl.when(c != 0)
            def _():
                pltpu.sync_copy(zeros_hbm.at[pl.ds(s * rpt, rpt)],
                                acc.at[pl.ds(s * rpt, rpt)])
        else:
            pltpu.sync_copy(g_hbm.at[pl.ds(c * n + s * rpt, rpt)],
                            acc.at[pl.ds(s * rpt, rpt)])
        plsc.subcore_barrier()

        def issue_idx(bi, t):
            srcb_b, _, dstb_b, _, isem_b, _ = bufs[bi]
            off = pl.multiple_of((w * K + t) * CHUNK, CHUNK)
            pltpu.async_copy(dst_hbm.at[pl.ds(off, CHUNK)], dstb_b, isem_b)
            if not const_rows:
                pltpu.async_copy(src_hbm.at[pl.ds(off, CHUNK)], srcb_b,
                                 isem_b)

        def wait_idx(bi):
            srcb_b, _, dstb_b, _, isem_b, _ = bufs[bi]
            pltpu.make_async_copy(dst_hbm.at[pl.ds(0, CHUNK)], dstb_b,
                                  isem_b).wait()
            if not const_rows:
                pltpu.make_async_copy(src_hbm.at[pl.ds(0, CHUNK)], srcb_b,
                                      isem_b).wait()

        def adjust_and_gather(bi):
            srcb_b, srca_b, _, rows_b, _, gsem_b = bufs[bi]
            for j in range(CHUNK // 16):
                sl = pl.ds(j * 16, 16)
                srca_b[sl] = srcb_b[sl] + base
            pltpu.async_copy(g_hbm.at[srca_b], rows_b, gsem_b)

        if const_rows:
            issue_idx(0, 0)
            issue_idx(1, 1)

            def body(i, carry):
                for b in (0, 1):
                    _, _, dstb_b, _, _, _ = bufs[b]
                    wait_idx(b)
                    pltpu.sync_copy(rows0, acc.at[dstb_b], add=True)

                    @pl.when(2 * i + b + 2 < K)
                    def _():
                        issue_idx(b, 2 * i + b + 2)
                return carry

            lax.fori_loop(0, K // 2, body, 0)
        else:
            issue_idx(0, 0)
            issue_idx(1, 1)
            wait_idx(0)
            adjust_and_gather(0)

            def body(i, carry):
                for b in (0, 1):
                    _, _, dstb_b, rows_b, _, gsem_b = bufs[b]
                    pltpu.make_async_copy(g_hbm.at[bufs[b][1]], rows_b,
                                          gsem_b).wait()
                    pltpu.sync_copy(rows_b, acc.at[dstb_b], add=True)

                    @pl.when(2 * i + b + 2 < K)
                    def _():
                        issue_idx(b, 2 * i + b + 2)

                    b2 = 1 - b
                    @pl.when(2 * i + b + 1 < K)
                    def _():
                        wait_idx(b2)
                        adjust_and_gather(b2)
                return carry

            lax.fori_loop(0, K // 2, body, 0)

        plsc.subcore_barrier()
        pltpu.sync_copy(acc.at[pl.ds(s * rpt, rpt)],
                        out_hbm.at[c, pl.ds(s * rpt, rpt)])

    return layer_kernel


def _dis_from_hist(hist_blk):
    deg = 1.0 + hist_blk[0, :, 0:1] + hist_blk[1, :, 0:1]
    return lax.rsqrt(deg)


def _tc1_body(x_ref, w_ref, hist_ref, o_ref):
    dis = _dis_from_hist(hist_ref[...])
    o_ref[...] = dis * jnp.dot(x_ref[...], w_ref[...],
                               preferred_element_type=jnp.float32)


def _tc_mid_body(acc_ref, b_ref, w_ref, hist_ref, o_ref):
    dis = _dis_from_hist(hist_ref[...])
    a = jnp.concatenate([acc_ref[0], acc_ref[1]], axis=1)
    h = jax.nn.relu(dis * a + b_ref[...])
    o_ref[...] = dis * jnp.dot(h, w_ref[...],
                               preferred_element_type=jnp.float32)


def _tc_head_body(acc_ref, b_ref, wr_ref, br_ref, hist_ref, h_ref, r_ref):
    dis = _dis_from_hist(hist_ref[...])
    a = acc_ref[0] + acc_ref[1]
    h = jax.nn.relu(dis * a + b_ref[...])
    h_ref[...] = h
    r_ref[...] = jnp.dot(h, wr_ref[...],
                         preferred_element_type=jnp.float32) + br_ref[...]


def kernel(x, edge_index, W1, b1, W2, b2, W3, b3, Wr, br):
    n, d_in = x.shape
    e = edge_index.shape[1]
    dh = W1.shape[1]
    dhh = dh // 2
    dout = W3.shape[1]

    src = edge_index[0]
    dst = edge_index[1]
    npad = 10240
    egrain = 64 * CHUNK
    epad = ((e + egrain - 1) // egrain) * egrain
    if epad != e:
        fill = jnp.full((epad - e,), npad - 1, dtype=src.dtype)
        src = jnp.concatenate([src, fill])
        dst = jnp.concatenate([dst, fill])
    e = epad
    zeros_nh = jnp.zeros((npad, dout), jnp.float32)
    ones_ch = jnp.ones((CHUNK, dout), jnp.float32)

    hist = _make_layer_kernel(npad, e, dout, True, const_rows=True)(
        ones_ch, src, dst, zeros_nh)

    bm = 1024
    nb = npad // bm
    hist_spec = pl.BlockSpec((NC, bm, dout), lambda h, i: (0, i, 0))

    g1 = pl.pallas_call(
        _tc1_body,
        grid=(2, nb),
        in_specs=[
            pl.BlockSpec((bm, d_in), lambda h, i: (i, 0)),
            pl.BlockSpec((d_in, dhh), lambda h, i: (0, h)),
            hist_spec,
        ],
        out_specs=pl.BlockSpec((bm, dhh), lambda h, i: (h * nb + i, 0)),
        out_shape=jax.ShapeDtypeStruct((2 * npad, dhh), jnp.float32),
    )(x, W1, hist)

    acc1 = _make_layer_kernel(npad, e, dhh, False)(g1, src, dst, zeros_nh)

    g2 = pl.pallas_call(
        _tc_mid_body,
        grid=(2, nb),
        in_specs=[
            pl.BlockSpec((NC, bm, dhh), lambda h, i: (0, i, 0)),
            pl.BlockSpec((1, dh), lambda h, i: (0, 0)),
            pl.BlockSpec((dh, dhh), lambda h, i: (0, h)),
            hist_spec,
        ],
        out_specs=pl.BlockSpec((bm, dhh), lambda h, i: (h * nb + i, 0)),
        out_shape=jax.ShapeDtypeStruct((2 * npad, dhh), jnp.float32),
    )(acc1, b1.reshape(1, dh), W2, hist)
    acc2 = _make_layer_kernel(npad, e, dhh, False)(g2, src, dst, zeros_nh)

    g3 = pl.pallas_call(
        _tc_mid_body,
        grid=(nb,),
        in_specs=[
            pl.BlockSpec((NC, bm, dhh), lambda i: (0, i, 0)),
            pl.BlockSpec((1, dh), lambda i: (0, 0)),
            pl.BlockSpec((dh, dout), lambda i: (0, 0)),
            pl.BlockSpec((NC, bm, dout), lambda i: (0, i, 0)),
        ],
        out_specs=pl.BlockSpec((bm, dout), lambda i: (i, 0)),
        out_shape=jax.ShapeDtypeStruct((npad, dout), jnp.float32),
    )(acc2, b2.reshape(1, dh), W3, hist)
    acc3 = _make_layer_kernel(npad, e, dout, True)(g3, src, dst, zeros_nh)

    h3, recon = pl.pallas_call(
        _tc_head_body,
        grid=(nb,),
        in_specs=[
            pl.BlockSpec((NC, bm, dout), lambda i: (0, i, 0)),
            pl.BlockSpec((1, dout), lambda i: (0, 0)),
            pl.BlockSpec((dout, d_in), lambda i: (0, 0)),
            pl.BlockSpec((1, d_in), lambda i: (0, 0)),
            pl.BlockSpec((NC, bm, dout), lambda i: (0, i, 0)),
        ],
        out_specs=[
            pl.BlockSpec((bm, dout), lambda i: (i, 0)),
            pl.BlockSpec((bm, d_in), lambda i: (i, 0)),
        ],
        out_shape=[
            jax.ShapeDtypeStruct((n, dout), jnp.float32),
            jax.ShapeDtypeStruct((n, d_in), jnp.float32),
        ],
    )(acc3, b3.reshape(1, dout), Wr, br.reshape(1, d_in), hist)

    return (h3, recon)


# speedup vs baseline: 1.0949x; 1.0949x over previous
import functools

import jax
import jax.numpy as jnp
from jax import lax
from jax.experimental import pallas as pl
from jax.experimental.pallas import tpu as pltpu
from jax.experimental.pallas import tpu_sc as plsc

NC = 2
NS = 16
CHUNK = 128

_MESH = dict(core_axis_name="c", subcore_axis_name="s", num_cores=NC,
             num_subcores=NS)


def _make_layer_kernel(n, e, dh, edge_split, const_rows=False):
    nchunks = e // CHUNK
    rpt = n // NS
    nw = NC * NS if (edge_split or const_rows) else NS
    K = nchunks // nw
    assert K * nw == nchunks and K >= 4 and K % 2 == 0

    scratch = [
        pltpu.VMEM((CHUNK,), jnp.int32),
        pltpu.VMEM((CHUNK,), jnp.int32),
        pltpu.VMEM((CHUNK,), jnp.int32),
        pltpu.VMEM((CHUNK,), jnp.int32),
        pltpu.VMEM((CHUNK,), jnp.int32),
        pltpu.VMEM((CHUNK,), jnp.int32),
        pltpu.VMEM((CHUNK,), jnp.int32),
        pltpu.VMEM((CHUNK,), jnp.int32),
        pltpu.VMEM((CHUNK, dh), jnp.float32),
        pltpu.VMEM((CHUNK, dh), jnp.float32),
        pltpu.VMEM_SHARED((n, dh), jnp.float32),
        pltpu.SemaphoreType.DMA,
        pltpu.SemaphoreType.DMA,
        pltpu.SemaphoreType.DMA,
        pltpu.SemaphoreType.DMA,
        pltpu.SemaphoreType.DMA,
        pltpu.SemaphoreType.DMA,
    ]

    @functools.partial(
        pl.kernel,
        out_type=jax.ShapeDtypeStruct((NC, n, dh), jnp.float32),
        mesh=plsc.VectorSubcoreMesh(**_MESH),
        scratch_types=scratch,
    )
    def layer_kernel(g_hbm, src_hbm, dst_hbm, zeros_hbm, out_hbm,
                     srcb0, srcb1, srca0, srca1, dstb0, dstb1, dsts0, dsts1,
                     rows0, rows1, acc, isem0, isem1, gsem0, gsem1, ssem0,
                     ssem1):
        c = lax.axis_index("c")
        s = lax.axis_index("s")
        if edge_split or const_rows:
            w = c * NS + s
        else:
            w = s
        base = jnp.int32(0) if (edge_split or const_rows) else c * n
        bufs = ((srcb0, srca0, dstb0, dsts0, rows0, isem0, gsem0, ssem0),
                (srcb1, srca1, dstb1, dsts1, rows1, isem1, gsem1, ssem1))

        if const_rows:
            pltpu.sync_copy(g_hbm, rows0)
            pltpu.sync_copy(zeros_hbm.at[pl.ds(s * rpt, rpt)],
                            acc.at[pl.ds(s * rpt, rpt)])
        elif edge_split:
            @pl.when(c == 0)
            def _():
                pltpu.sync_copy(g_hbm.at[pl.ds(s * rpt, rpt)],
                                acc.at[pl.ds(s * rpt, rpt)])

            @pl.when(c != 0)
            def _():
                pltpu.sync_copy(zeros_hbm.at[pl.ds(s * rpt, rpt)],
                                acc.at[pl.ds(s * rpt, rpt)])
        else:
            pltpu.sync_copy(g_hbm.at[pl.ds(c * n + s * rpt, rpt)],
                            acc.at[pl.ds(s * rpt, rpt)])
        plsc.subcore_barrier()

        def issue_idx(bi, t):
            srcb_b, _, dstb_b, _, _, isem_b, _, _ = bufs[bi]
            off = pl.multiple_of((w * K + t) * CHUNK, CHUNK)
            pltpu.async_copy(dst_hbm.at[pl.ds(off, CHUNK)], dstb_b, isem_b)
            if not const_rows:
                pltpu.async_copy(src_hbm.at[pl.ds(off, CHUNK)], srcb_b,
                                 isem_b)

        def wait_idx(bi):
            srcb_b, _, dstb_b, _, _, isem_b, _, _ = bufs[bi]
            pltpu.make_async_copy(dst_hbm.at[pl.ds(0, CHUNK)], dstb_b,
                                  isem_b).wait()
            if not const_rows:
                pltpu.make_async_copy(src_hbm.at[pl.ds(0, CHUNK)], srcb_b,
                                      isem_b).wait()

        def adjust_and_gather(bi):
            srcb_b, srca_b, _, _, rows_b, _, gsem_b, _ = bufs[bi]
            for j in range(CHUNK // 16):
                sl = pl.ds(j * 16, 16)
                srca_b[sl] = srcb_b[sl] + base
            pltpu.async_copy(g_hbm.at[srca_b], rows_b, gsem_b)

        def wait_gather(bi):
            _, srca_b, _, _, rows_b, _, gsem_b, _ = bufs[bi]
            pltpu.make_async_copy(g_hbm.at[srca_b], rows_b, gsem_b).wait()

        def wait_scatter(bi):
            _, _, _, dsts_b, rows_b, _, _, ssem_b = bufs[bi]
            pltpu.make_async_copy(rows_b, acc.at[dsts_b], ssem_b).wait()

        if const_rows:
            issue_idx(0, 0)
            issue_idx(1, 1)

            def body(i, carry):
                for b in (0, 1):
                    dstb_b = bufs[b][2]
                    wait_idx(b)
                    pltpu.sync_copy(rows0, acc.at[dstb_b], add=True)
                    issue_idx(b, 2 * i + b + 2)
                return carry

            lax.fori_loop(0, K // 2, body, 0)
            wait_idx(0)
            wait_idx(1)
        else:
            issue_idx(0, 0)
            issue_idx(1, 1)
            wait_idx(0)
            adjust_and_gather(0)
            for j in range(CHUNK // 16):
                dsts1[pl.ds(j * 16, 16)] = jnp.full((16,), n - 1, jnp.int32)
            pltpu.async_copy(rows1, acc.at[dsts1], ssem1, add=True)

            def body(i, carry):
                for b in (0, 1):
                    b2 = 1 - b
                    _, _, dstb_b, dsts_b, rows_b, _, _, ssem_b = bufs[b]
                    wait_gather(b)
                    for j in range(CHUNK // 16):
                        sl = pl.ds(j * 16, 16)
                        dsts_b[sl] = dstb_b[sl]
                    pltpu.async_copy(rows_b, acc.at[dsts_b], ssem_b,
                                     add=True)
                    issue_idx(b, 2 * i + b + 2)
                    wait_idx(b2)
                    wait_scatter(b2)
                    adjust_and_gather(b2)
                return carry

            lax.fori_loop(0, K // 2, body, 0)
            wait_gather(0)
            wait_idx(1)
            wait_scatter(1)

        plsc.subcore_barrier()
        pltpu.sync_copy(acc.at[pl.ds(s * rpt, rpt)],
                        out_hbm.at[c, pl.ds(s * rpt, rpt)])

    return layer_kernel


def _dis_from_hist(hist_blk):
    deg = 1.0 + hist_blk[0, :, 0:1] + hist_blk[1, :, 0:1]
    return lax.rsqrt(deg)


def _tc1_body(x_ref, w_ref, hist_ref, o_ref):
    dis = _dis_from_hist(hist_ref[...])
    o_ref[...] = dis * jnp.dot(x_ref[...], w_ref[...],
                               preferred_element_type=jnp.float32)


def _tc_mid_body(acc_ref, b_ref, w_ref, hist_ref, o_ref):
    dis = _dis_from_hist(hist_ref[...])
    a = jnp.concatenate([acc_ref[0], acc_ref[1]], axis=1)
    h = jax.nn.relu(dis * a + b_ref[...])
    o_ref[...] = dis * jnp.dot(h, w_ref[...],
                               preferred_element_type=jnp.float32)


def _tc_head_body(acc_ref, b_ref, wr_ref, br_ref, hist_ref, h_ref, r_ref):
    dis = _dis_from_hist(hist_ref[...])
    a = acc_ref[0] + acc_ref[1]
    h = jax.nn.relu(dis * a + b_ref[...])
    h_ref[...] = h
    r_ref[...] = jnp.dot(h, wr_ref[...],
                         preferred_element_type=jnp.float32) + br_ref[...]


def kernel(x, edge_index, W1, b1, W2, b2, W3, b3, Wr, br):
    n, d_in = x.shape
    e = edge_index.shape[1]
    dh = W1.shape[1]
    dhh = dh // 2
    dout = W3.shape[1]

    src = edge_index[0]
    dst = edge_index[1]
    npad = 10240
    egrain = 64 * CHUNK
    epad = ((e + egrain - 1) // egrain) * egrain
    fill = jnp.full((epad + 2 * CHUNK - e,), npad - 1, dtype=src.dtype)
    src = jnp.concatenate([src, fill])
    dst = jnp.concatenate([dst, fill])
    e = epad
    zeros_nh = jnp.zeros((npad, dout), jnp.float32)
    ones_ch = jnp.ones((CHUNK, dout), jnp.float32)

    hist = _make_layer_kernel(npad, e, dout, True, const_rows=True)(
        ones_ch, src, dst, zeros_nh)

    bm = 1024
    nb = npad // bm
    hist_spec = pl.BlockSpec((NC, bm, dout), lambda h, i: (0, i, 0))

    g1 = pl.pallas_call(
        _tc1_body,
        grid=(2, nb),
        in_specs=[
            pl.BlockSpec((bm, d_in), lambda h, i: (i, 0)),
            pl.BlockSpec((d_in, dhh), lambda h, i: (0, h)),
            hist_spec,
        ],
        out_specs=pl.BlockSpec((bm, dhh), lambda h, i: (h * nb + i, 0)),
        out_shape=jax.ShapeDtypeStruct((2 * npad, dhh), jnp.float32),
    )(x, W1, hist)

    acc1 = _make_layer_kernel(npad, e, dhh, False)(g1, src, dst, zeros_nh)

    g2 = pl.pallas_call(
        _tc_mid_body,
        grid=(2, nb),
        in_specs=[
            pl.BlockSpec((NC, bm, dhh), lambda h, i: (0, i, 0)),
            pl.BlockSpec((1, dh), lambda h, i: (0, 0)),
            pl.BlockSpec((dh, dhh), lambda h, i: (0, h)),
            hist_spec,
        ],
        out_specs=pl.BlockSpec((bm, dhh), lambda h, i: (h * nb + i, 0)),
        out_shape=jax.ShapeDtypeStruct((2 * npad, dhh), jnp.float32),
    )(acc1, b1.reshape(1, dh), W2, hist)
    acc2 = _make_layer_kernel(npad, e, dhh, False)(g2, src, dst, zeros_nh)

    g3 = pl.pallas_call(
        _tc_mid_body,
        grid=(nb,),
        in_specs=[
            pl.BlockSpec((NC, bm, dhh), lambda i: (0, i, 0)),
            pl.BlockSpec((1, dh), lambda i: (0, 0)),
            pl.BlockSpec((dh, dout), lambda i: (0, 0)),
            pl.BlockSpec((NC, bm, dout), lambda i: (0, i, 0)),
        ],
        out_specs=pl.BlockSpec((bm, dout), lambda i: (i, 0)),
        out_shape=jax.ShapeDtypeStruct((npad, dout), jnp.float32),
    )(acc2, b2.reshape(1, dh), W3, hist)
    acc3 = _make_layer_kernel(npad, e, dout, True)(g3, src, dst, zeros_nh)

    h3, recon = pl.pallas_call(
        _tc_head_body,
        grid=(nb,),
        in_specs=[
            pl.BlockSpec((NC, bm, dout), lambda i: (0, i, 0)),
            pl.BlockSpec((1, dout), lambda i: (0, 0)),
            pl.BlockSpec((dout, d_in), lambda i: (0, 0)),
            pl.BlockSpec((1, d_in), lambda i: (0, 0)),
            pl.BlockSpec((NC, bm, dout), lambda i: (0, i, 0)),
        ],
        out_specs=[
            pl.BlockSpec((bm, dout), lambda i: (i, 0)),
            pl.BlockSpec((bm, d_in), lambda i: (i, 0)),
        ],
        out_shape=[
            jax.ShapeDtypeStruct((n, dout), jnp.float32),
            jax.ShapeDtypeStruct((n, d_in), jnp.float32),
        ],
    )(acc3, b3.reshape(1, dout), Wr, br.reshape(1, d_in), hist)

    return (h3, recon)


# speedup vs baseline: 1.0953x; 1.0004x over previous
import functools

import jax
import jax.numpy as jnp
from jax import lax
from jax.experimental import pallas as pl
from jax.experimental.pallas import tpu as pltpu
from jax.experimental.pallas import tpu_sc as plsc

NC = 2
NS = 16
CHUNK = 128

_MESH = dict(core_axis_name="c", subcore_axis_name="s", num_cores=NC,
             num_subcores=NS)


def _make_layer_kernel(n, e, dh, edge_split, const_rows=False):
    nchunks = e // CHUNK
    rpt = n // NS
    nw = NC * NS if (edge_split or const_rows) else NS
    K = nchunks // nw
    assert K * nw == nchunks and K >= 4 and K % 2 == 0

    scratch = [
        pltpu.VMEM((CHUNK,), jnp.int32),
        pltpu.VMEM((CHUNK,), jnp.int32),
        pltpu.VMEM((CHUNK,), jnp.int32),
        pltpu.VMEM((CHUNK,), jnp.int32),
        pltpu.VMEM((CHUNK,), jnp.int32),
        pltpu.VMEM((CHUNK,), jnp.int32),
        pltpu.VMEM((CHUNK,), jnp.int32),
        pltpu.VMEM((CHUNK,), jnp.int32),
        pltpu.VMEM((CHUNK, dh), jnp.float32),
        pltpu.VMEM((CHUNK, dh), jnp.float32),
        pltpu.VMEM_SHARED((n, dh), jnp.float32),
        pltpu.SemaphoreType.DMA,
        pltpu.SemaphoreType.DMA,
        pltpu.SemaphoreType.DMA,
        pltpu.SemaphoreType.DMA,
        pltpu.SemaphoreType.DMA,
        pltpu.SemaphoreType.DMA,
    ]

    @functools.partial(
        pl.kernel,
        out_type=jax.ShapeDtypeStruct((NC, n, dh), jnp.float32),
        mesh=plsc.VectorSubcoreMesh(**_MESH),
        scratch_types=scratch,
    )
    def layer_kernel(g_hbm, src_hbm, dst_hbm, zeros_hbm, out_hbm,
                     srcb0, srcb1, srca0, srca1, dstb0, dstb1, dsts0, dsts1,
                     rows0, rows1, acc, isem0, isem1, gsem0, gsem1, ssem0,
                     ssem1):
        c = lax.axis_index("c")
        s = lax.axis_index("s")
        if edge_split or const_rows:
            w = c * NS + s
        else:
            w = s
        base = jnp.int32(0) if (edge_split or const_rows) else c * n
        bufs = ((srcb0, srca0, dstb0, dsts0, rows0, isem0, gsem0, ssem0),
                (srcb1, srca1, dstb1, dsts1, rows1, isem1, gsem1, ssem1))

        if const_rows:
            pltpu.sync_copy(g_hbm, rows0)
            pltpu.sync_copy(zeros_hbm.at[pl.ds(s * rpt, rpt)],
                            acc.at[pl.ds(s * rpt, rpt)])
        elif edge_split:
            @pl.when(c == 0)
            def _():
                pltpu.sync_copy(g_hbm.at[pl.ds(s * rpt, rpt)],
                                acc.at[pl.ds(s * rpt, rpt)])

            @pl.when(c != 0)
            def _():
                pltpu.sync_copy(zeros_hbm.at[pl.ds(s * rpt, rpt)],
                                acc.at[pl.ds(s * rpt, rpt)])
        else:
            pltpu.sync_copy(g_hbm.at[pl.ds(c * n + s * rpt, rpt)],
                            acc.at[pl.ds(s * rpt, rpt)])
        plsc.subcore_barrier()

        def issue_idx(bi, t):
            srcb_b, _, dstb_b, _, _, isem_b, _, _ = bufs[bi]
            off = pl.multiple_of((w * K + t) * CHUNK, CHUNK)
            pltpu.async_copy(dst_hbm.at[pl.ds(off, CHUNK)], dstb_b, isem_b)
            if not const_rows:
                pltpu.async_copy(src_hbm.at[pl.ds(off, CHUNK)], srcb_b,
                                 isem_b)

        def wait_idx(bi):
            srcb_b, _, dstb_b, _, _, isem_b, _, _ = bufs[bi]
            pltpu.make_async_copy(dst_hbm.at[pl.ds(0, CHUNK)], dstb_b,
                                  isem_b).wait()
            if not const_rows:
                pltpu.make_async_copy(src_hbm.at[pl.ds(0, CHUNK)], srcb_b,
                                      isem_b).wait()

        def adjust_and_gather(bi):
            srcb_b, srca_b, _, _, rows_b, _, gsem_b, _ = bufs[bi]
            for j in range(CHUNK // 16):
                sl = pl.ds(j * 16, 16)
                srca_b[sl] = srcb_b[sl] + base
            pltpu.async_copy(g_hbm.at[srca_b], rows_b, gsem_b)

        def wait_gather(bi):
            _, _, _, _, rows_b, _, gsem_b, _ = bufs[bi]
            pltpu.make_async_copy(g_hbm.at[pl.ds(0, CHUNK)], rows_b,
                                  gsem_b).wait()

        def wait_scatter(bi):
            _, _, _, _, rows_b, _, _, ssem_b = bufs[bi]
            pltpu.make_async_copy(g_hbm.at[pl.ds(0, CHUNK)], rows_b,
                                  ssem_b).wait()

        if const_rows:
            issue_idx(0, 0)
            issue_idx(1, 1)

            def body(i, carry):
                for b in (0, 1):
                    dstb_b = bufs[b][2]
                    wait_idx(b)
                    pltpu.sync_copy(rows0, acc.at[dstb_b], add=True)
                    issue_idx(b, 2 * i + b + 2)
                return carry

            lax.fori_loop(0, K // 2, body, 0)
            wait_idx(0)
            wait_idx(1)
        else:
            issue_idx(0, 0)
            issue_idx(1, 1)
            wait_idx(0)
            adjust_and_gather(0)
            for j in range(CHUNK // 16):
                dsts1[pl.ds(j * 16, 16)] = jnp.full((16,), n - 1, jnp.int32)
            pltpu.async_copy(rows1, acc.at[dsts1], ssem1, add=True)

            def body(i, carry):
                for b in (0, 1):
                    b2 = 1 - b
                    _, _, dstb_b, dsts_b, rows_b, _, _, ssem_b = bufs[b]
                    wait_gather(b)
                    for j in range(CHUNK // 16):
                        sl = pl.ds(j * 16, 16)
                        dsts_b[sl] = dstb_b[sl]
                    pltpu.async_copy(rows_b, acc.at[dsts_b], ssem_b,
                                     add=True)
                    issue_idx(b, 2 * i + b + 2)
                    wait_idx(b2)
                    wait_scatter(b2)
                    adjust_and_gather(b2)
                return carry

            lax.fori_loop(0, K // 2, body, 0)
            wait_gather(0)
            wait_idx(1)
            wait_scatter(1)

        plsc.subcore_barrier()
        pltpu.sync_copy(acc.at[pl.ds(s * rpt, rpt)],
                        out_hbm.at[c, pl.ds(s * rpt, rpt)])

    return layer_kernel


def _dis_from_hist(hist_blk):
    deg = 1.0 + hist_blk[0, :, 0:1] + hist_blk[1, :, 0:1]
    return lax.rsqrt(deg)


def _tc1_body(x_ref, w_ref, hist_ref, o_ref):
    dis = _dis_from_hist(hist_ref[...])
    o_ref[...] = dis * jnp.dot(x_ref[...], w_ref[...],
                               preferred_element_type=jnp.float32)


def _tc_mid_body(acc_ref, b_ref, w_ref, hist_ref, o_ref):
    dis = _dis_from_hist(hist_ref[...])
    a = jnp.concatenate([acc_ref[0], acc_ref[1]], axis=1)
    h = jax.nn.relu(dis * a + b_ref[...])
    o_ref[...] = dis * jnp.dot(h, w_ref[...],
                               preferred_element_type=jnp.float32)


def _tc_head_body(acc_ref, b_ref, wr_ref, br_ref, hist_ref, h_ref, r_ref):
    dis = _dis_from_hist(hist_ref[...])
    a = acc_ref[0] + acc_ref[1]
    h = jax.nn.relu(dis * a + b_ref[...])
    h_ref[...] = h
    r_ref[...] = jnp.dot(h, wr_ref[...],
                         preferred_element_type=jnp.float32) + br_ref[...]


def kernel(x, edge_index, W1, b1, W2, b2, W3, b3, Wr, br):
    n, d_in = x.shape
    e = edge_index.shape[1]
    dh = W1.shape[1]
    dhh = dh // 2
    dout = W3.shape[1]

    src = edge_index[0]
    dst = edge_index[1]
    npad = 10240
    egrain = 64 * CHUNK
    epad = ((e + egrain - 1) // egrain) * egrain
    fill = jnp.full((epad + 2 * CHUNK - e,), npad - 1, dtype=src.dtype)
    src = jnp.concatenate([src, fill])
    dst = jnp.concatenate([dst, fill])
    e = epad
    zeros_nh = jnp.zeros((npad, dout), jnp.float32)
    ones_ch = jnp.ones((CHUNK, dout), jnp.float32)

    hist = _make_layer_kernel(npad, e, dout, True, const_rows=True)(
        ones_ch, src, dst, zeros_nh)

    bm = 1024
    nb = npad // bm
    hist_spec = pl.BlockSpec((NC, bm, dout), lambda h, i: (0, i, 0))

    g1 = pl.pallas_call(
        _tc1_body,
        grid=(2, nb),
        in_specs=[
            pl.BlockSpec((bm, d_in), lambda h, i: (i, 0)),
            pl.BlockSpec((d_in, dhh), lambda h, i: (0, h)),
            hist_spec,
        ],
        out_specs=pl.BlockSpec((bm, dhh), lambda h, i: (h * nb + i, 0)),
        out_shape=jax.ShapeDtypeStruct((2 * npad, dhh), jnp.float32),
    )(x, W1, hist)

    acc1 = _make_layer_kernel(npad, e, dhh, False)(g1, src, dst, zeros_nh)

    g2 = pl.pallas_call(
        _tc_mid_body,
        grid=(2, nb),
        in_specs=[
            pl.BlockSpec((NC, bm, dhh), lambda h, i: (0, i, 0)),
            pl.BlockSpec((1, dh), lambda h, i: (0, 0)),
            pl.BlockSpec((dh, dhh), lambda h, i: (0, h)),
            hist_spec,
        ],
        out_specs=pl.BlockSpec((bm, dhh), lambda h, i: (h * nb + i, 0)),
        out_shape=jax.ShapeDtypeStruct((2 * npad, dhh), jnp.float32),
    )(acc1, b1.reshape(1, dh), W2, hist)
    acc2 = _make_layer_kernel(npad, e, dhh, False)(g2, src, dst, zeros_nh)

    g3 = pl.pallas_call(
        _tc_mid_body,
        grid=(nb,),
        in_specs=[
            pl.BlockSpec((NC, bm, dhh), lambda i: (0, i, 0)),
            pl.BlockSpec((1, dh), lambda i: (0, 0)),
            pl.BlockSpec((dh, dout), lambda i: (0, 0)),
            pl.BlockSpec((NC, bm, dout), lambda i: (0, i, 0)),
        ],
        out_specs=pl.BlockSpec((bm, dout), lambda i: (i, 0)),
        out_shape=jax.ShapeDtypeStruct((npad, dout), jnp.float32),
    )(acc2, b2.reshape(1, dh), W3, hist)
    acc3 = _make_layer_kernel(npad, e, dout, True)(g3, src, dst, zeros_nh)

    h3, recon = pl.pallas_call(
        _tc_head_body,
        grid=(nb,),
        in_specs=[
            pl.BlockSpec((NC, bm, dout), lambda i: (0, i, 0)),
            pl.BlockSpec((1, dout), lambda i: (0, 0)),
            pl.BlockSpec((dout, d_in), lambda i: (0, 0)),
            pl.BlockSpec((1, d_in), lambda i: (0, 0)),
            pl.BlockSpec((NC, bm, dout), lambda i: (0, i, 0)),
        ],
        out_specs=[
            pl.BlockSpec((bm, dout), lambda i: (i, 0)),
            pl.BlockSpec((bm, d_in), lambda i: (i, 0)),
        ],
        out_shape=[
            jax.ShapeDtypeStruct((n, dout), jnp.float32),
            jax.ShapeDtypeStruct((n, d_in), jnp.float32),
        ],
    )(acc3, b3.reshape(1, dout), Wr, br.reshape(1, d_in), hist)

    return (h3, recon)
